# baseline (speedup 1.0000x reference)
import functools

import jax
import jax.numpy as jnp
import numpy as np
from jax import lax
from jax.experimental import pallas as pl
from jax.experimental.pallas import tpu as pltpu
from jax.experimental.pallas import tpu_sc as plsc

N = 10000
E = 320000
D = 128

NC = 2
NS = 16
NW = NC * NS

CHUNK = 64
CPW = 160
E_PAD = NW * CPW * CHUNK
NBUF = 5
NIDX = 10
GL = 3
KI = NIDX - NBUF + GL
NTRASH = 96
ACC_ROWS = N + NTRASH

ROWS_PER_SUB = 624
TAIL_ROWS = N - NS * ROWS_PER_SUB


def _seg_sum_body(e_hbm, x_hbm, zeros_hbm, out_hbm,
                  idx, rows, acc, semi, semg, sems):
    c = lax.axis_index("c")
    s = lax.axis_index("s")
    w = s * NC + c

    @pl.when(c == 0)
    def _():
        pltpu.sync_copy(x_hbm.at[pl.ds(s * ROWS_PER_SUB, ROWS_PER_SUB)],
                        acc.at[pl.ds(s * ROWS_PER_SUB, ROWS_PER_SUB)])

    @pl.when(c == 1)
    def _():
        pltpu.sync_copy(zeros_hbm,
                        acc.at[pl.ds(s * ROWS_PER_SUB, ROWS_PER_SUB)])

    @pl.when((s == 0) & (c == 0))
    def _():
        pltpu.sync_copy(x_hbm.at[pl.ds(NS * ROWS_PER_SUB, TAIL_ROWS)],
                        acc.at[pl.ds(NS * ROWS_PER_SUB, TAIL_ROWS)])

    @pl.when((s == 0) & (c == 1))
    def _():
        pltpu.sync_copy(zeros_hbm.at[pl.ds(0, TAIL_ROWS)],
                        acc.at[pl.ds(NS * ROWS_PER_SUB, TAIL_ROWS)])

    @pl.when(s == 1)
    def _():
        pltpu.sync_copy(zeros_hbm.at[pl.ds(0, NTRASH)],
                        acc.at[pl.ds(N, NTRASH)])

    plsc.subcore_barrier()

    def start_idx(j, q):
        pltpu.async_copy(e_hbm.at[w, j], idx.at[q], semi[q])

    def wait_idx(j, q):
        pltpu.make_async_copy(e_hbm.at[w, j], idx.at[q], semi[q]).wait()

    def start_gather(q, b):
        pltpu.async_copy(x_hbm.at[idx.at[q, 0]], rows.at[b], semg[b])

    def wait_gather(q, b):
        pltpu.make_async_copy(x_hbm.at[idx.at[q, 0]], rows.at[b],
                              semg[b]).wait()

    def start_scatter(q, b):
        pltpu.async_copy(rows.at[b], acc.at[idx.at[q, 1]], sems[b], add=True)

    def wait_scatter(q, b):
        pltpu.make_async_copy(rows.at[b], acc.at[idx.at[q, 1]],
                              sems[b]).wait()

    for q in range(KI):
        start_idx(q, q)
    for q in range(GL):
        wait_idx(q, q)
        start_gather(q, q)

    def slot(j, u, dyn):
        if dyn or j + GL < CPW:
            wait_idx(j + GL, (u + GL) % NIDX)
            if dyn or j - (NBUF - GL) >= 0:
                wait_scatter((u - (NBUF - GL)) % NIDX, (u + GL) % NBUF)
            start_gather((u + GL) % NIDX, (u + GL) % NBUF)
        if dyn or j + KI < CPW:
            start_idx(j + KI, (u + KI) % NIDX)
        wait_gather(u % NIDX, u % NBUF)
        start_scatter(u % NIDX, u % NBUF)

    for u in range(NIDX):
        slot(u, u, dyn=False)

    def loop_body(p, carry):
        j = p * NIDX
        for u in range(NIDX):
            slot(j + u, u, dyn=True)
        return carry

    lax.fori_loop(1, CPW // NIDX - 1, loop_body, 0)

    for u in range(NIDX):
        slot(CPW - NIDX + u, u, dyn=False)

    for j in range(CPW - NBUF, CPW):
        wait_scatter(j % NIDX, j % NBUF)

    plsc.subcore_barrier()
    pltpu.sync_copy(acc.at[pl.ds(s * ROWS_PER_SUB, ROWS_PER_SUB)],
                    out_hbm.at[c, pl.ds(s * ROWS_PER_SUB, ROWS_PER_SUB)])

    @pl.when(s == 0)
    def _():
        pltpu.sync_copy(acc.at[pl.ds(NS * ROWS_PER_SUB, TAIL_ROWS)],
                        out_hbm.at[c, pl.ds(NS * ROWS_PER_SUB, TAIL_ROWS)])


def _seg_sum(e4d, x, zeros):
    mesh = plsc.VectorSubcoreMesh(core_axis_name="c", subcore_axis_name="s",
                                  num_cores=NC, num_subcores=NS)
    f = pl.kernel(
        _seg_sum_body,
        out_type=jax.ShapeDtypeStruct((NC, N, D), jnp.float32),
        mesh=mesh,
        scratch_types=[
            pltpu.VMEM((NIDX, 2, CHUNK), jnp.int32),
            pltpu.VMEM((NBUF, CHUNK, D), jnp.float32),
            pltpu.VMEM_SHARED((ACC_ROWS, D), jnp.float32),
            [pltpu.SemaphoreType.DMA] * NIDX,
            [pltpu.SemaphoreType.DMA] * NBUF,
            [pltpu.SemaphoreType.DMA] * NBUF,
        ],
    )
    return f(e4d, x, zeros)


def _mlp_bn_body(p_ref, Wa_ref, ba_ref, Wb_ref, bb_ref, g_ref, be_ref,
                 out_ref):
    h = p_ref[0] + p_ref[1]
    h = jnp.maximum(
        jnp.dot(h, Wa_ref[...], preferred_element_type=jnp.float32) + ba_ref[...], 0.0)
    h = jnp.dot(h, Wb_ref[...], preferred_element_type=jnp.float32) + bb_ref[...]
    mu = jnp.mean(h, axis=0, keepdims=True)
    var = jnp.mean((h - mu) * (h - mu), axis=0, keepdims=True)
    h = (h - mu) * lax.rsqrt(var + 1e-5) * g_ref[...] + be_ref[...]
    out_ref[...] = jnp.maximum(h, 0.0)


def _mlp_bn_res_body(p_ref, Wa_ref, ba_ref, Wb_ref, bb_ref, g_ref, be_ref,
                     x0_ref, out_ref):
    h = p_ref[0] + p_ref[1]
    h = jnp.maximum(
        jnp.dot(h, Wa_ref[...], preferred_element_type=jnp.float32) + ba_ref[...], 0.0)
    h = jnp.dot(h, Wb_ref[...], preferred_element_type=jnp.float32) + bb_ref[...]
    mu = jnp.mean(h, axis=0, keepdims=True)
    var = jnp.mean((h - mu) * (h - mu), axis=0, keepdims=True)
    h = (h - mu) * lax.rsqrt(var + 1e-5) * g_ref[...] + be_ref[...]
    out_ref[...] = (jnp.maximum(h, 0.0) + x0_ref[...]) * np.float32(1.0 / np.sqrt(2.0))


def _mlp_bn(p, Wa, ba, Wb, bb, g, be):
    return pl.pallas_call(
        _mlp_bn_body,
        out_shape=jax.ShapeDtypeStruct((N, D), jnp.float32),
    )(p, Wa, ba.reshape(1, D), Wb, bb.reshape(1, D), g.reshape(1, D),
      be.reshape(1, D))


def _mlp_bn_res(p, Wa, ba, Wb, bb, g, be, x0):
    return pl.pallas_call(
        _mlp_bn_res_body,
        out_shape=jax.ShapeDtypeStruct((N, D), jnp.float32),
    )(p, Wa, ba.reshape(1, D), Wb, bb.reshape(1, D), g.reshape(1, D),
      be.reshape(1, D), x0)


def kernel(x, edge_index, W1, b1, W2, b2, W3, b3, W4, b4, g1, be1, g2, be2):
    src = edge_index[0].astype(jnp.int32)
    dst = edge_index[1].astype(jnp.int32)
    pad_i = jnp.arange(E_PAD - E, dtype=jnp.int32)
    src3d = jnp.concatenate([src, pad_i % N]).reshape(NW, CPW, CHUNK)
    dst3d = jnp.concatenate([dst, N + pad_i % NTRASH]).reshape(NW, CPW, CHUNK)
    e4d = jnp.stack([src3d, dst3d], axis=2)
    zeros = jnp.zeros((ROWS_PER_SUB, D), jnp.float32)

    p1 = _seg_sum(e4d, x, zeros)
    h1 = _mlp_bn(p1, W1, b1, W2, b2, g1, be1)
    p2 = _seg_sum(e4d, h1, zeros)
    return _mlp_bn_res(p2, W3, b3, W4, b4, g2, be2, x)


# speedup vs baseline: 1.0686x; 1.0686x over previous
import functools

import jax
import jax.numpy as jnp
import numpy as np
from jax import lax
from jax.experimental import pallas as pl
from jax.experimental.pallas import tpu as pltpu
from jax.experimental.pallas import tpu_sc as plsc

N = 10000
E = 320000
D = 128

NC = 2
NS = 16
NW = NC * NS

CHUNK = 112
CPW = 90
E_PAD = NW * CPW * CHUNK
NBUF = 3
NIDX = 6
GL = 2
KI = NIDX - NBUF + GL
NTRASH = 96
ACC_ROWS = N + NTRASH

ROWS_PER_SUB = 624
TAIL_ROWS = N - NS * ROWS_PER_SUB


def _seg_sum_body(src_hbm, dst_hbm, x_hbm, zeros_hbm, out_hbm,
                  idx_s, idx_d, rows, acc, semi, semg, sems):
    c = lax.axis_index("c")
    s = lax.axis_index("s")
    w = s * NC + c

    @pl.when(c == 0)
    def _():
        pltpu.sync_copy(x_hbm.at[pl.ds(s * ROWS_PER_SUB, ROWS_PER_SUB)],
                        acc.at[pl.ds(s * ROWS_PER_SUB, ROWS_PER_SUB)])

    @pl.when(c == 1)
    def _():
        pltpu.sync_copy(zeros_hbm,
                        acc.at[pl.ds(s * ROWS_PER_SUB, ROWS_PER_SUB)])

    @pl.when((s == 0) & (c == 0))
    def _():
        pltpu.sync_copy(x_hbm.at[pl.ds(NS * ROWS_PER_SUB, TAIL_ROWS)],
                        acc.at[pl.ds(NS * ROWS_PER_SUB, TAIL_ROWS)])

    @pl.when((s == 0) & (c == 1))
    def _():
        pltpu.sync_copy(zeros_hbm.at[pl.ds(0, TAIL_ROWS)],
                        acc.at[pl.ds(NS * ROWS_PER_SUB, TAIL_ROWS)])

    @pl.when(s == 1)
    def _():
        pltpu.sync_copy(zeros_hbm.at[pl.ds(0, NTRASH)],
                        acc.at[pl.ds(N, NTRASH)])

    plsc.subcore_barrier()

    def start_idx(j, q):
        pltpu.async_copy(src_hbm.at[w, j], idx_s.at[q], semi[q])
        pltpu.async_copy(dst_hbm.at[w, j], idx_d.at[q], semi[q])

    def wait_idx(j, q):
        pltpu.make_async_copy(src_hbm.at[w, j], idx_s.at[q], semi[q]).wait()
        pltpu.make_async_copy(dst_hbm.at[w, j], idx_d.at[q], semi[q]).wait()

    def start_gather(q, b):
        pltpu.async_copy(x_hbm.at[idx_s.at[q]], rows.at[b], semg[b])

    def wait_gather(q, b):
        pltpu.make_async_copy(x_hbm.at[idx_s.at[q]], rows.at[b],
                              semg[b]).wait()

    def start_scatter(q, b):
        pltpu.async_copy(rows.at[b], acc.at[idx_d.at[q]], sems[b], add=True)

    def wait_scatter(q, b):
        pltpu.make_async_copy(rows.at[b], acc.at[idx_d.at[q]],
                              sems[b]).wait()

    for q in range(KI):
        start_idx(q, q)
    for q in range(GL):
        wait_idx(q, q)
        start_gather(q, q)

    def slot(j, u, dyn):
        if dyn or j + GL < CPW:
            wait_idx(j + GL, (u + GL) % NIDX)
            if dyn or j - (NBUF - GL) >= 0:
                wait_scatter((u - (NBUF - GL)) % NIDX, (u + GL) % NBUF)
            start_gather((u + GL) % NIDX, (u + GL) % NBUF)
        if dyn or j + KI < CPW:
            start_idx(j + KI, (u + KI) % NIDX)
        wait_gather(u % NIDX, u % NBUF)
        start_scatter(u % NIDX, u % NBUF)

    for u in range(NIDX):
        slot(u, u, dyn=False)

    def loop_body(p, carry):
        j = p * NIDX
        for u in range(NIDX):
            slot(j + u, u, dyn=True)
        return carry

    lax.fori_loop(1, CPW // NIDX - 1, loop_body, 0)

    for u in range(NIDX):
        slot(CPW - NIDX + u, u, dyn=False)

    for j in range(CPW - NBUF, CPW):
        wait_scatter(j % NIDX, j % NBUF)

    plsc.subcore_barrier()
    pltpu.sync_copy(acc.at[pl.ds(s * ROWS_PER_SUB, ROWS_PER_SUB)],
                    out_hbm.at[c, pl.ds(s * ROWS_PER_SUB, ROWS_PER_SUB)])

    @pl.when(s == 0)
    def _():
        pltpu.sync_copy(acc.at[pl.ds(NS * ROWS_PER_SUB, TAIL_ROWS)],
                        out_hbm.at[c, pl.ds(NS * ROWS_PER_SUB, TAIL_ROWS)])


def _seg_sum(src3d, dst3d, x, zeros):
    mesh = plsc.VectorSubcoreMesh(core_axis_name="c", subcore_axis_name="s",
                                  num_cores=NC, num_subcores=NS)
    f = pl.kernel(
        _seg_sum_body,
        out_type=jax.ShapeDtypeStruct((NC, N, D), jnp.float32),
        mesh=mesh,
        scratch_types=[
            pltpu.VMEM((NIDX, CHUNK), jnp.int32),
            pltpu.VMEM((NIDX, CHUNK), jnp.int32),
            pltpu.VMEM((NBUF, CHUNK, D), jnp.float32),
            pltpu.VMEM_SHARED((ACC_ROWS, D), jnp.float32),
            [pltpu.SemaphoreType.DMA] * NIDX,
            [pltpu.SemaphoreType.DMA] * NBUF,
            [pltpu.SemaphoreType.DMA] * NBUF,
        ],
    )
    return f(src3d, dst3d, x, zeros)


def _mlp_bn_body(p_ref, Wa_ref, ba_ref, Wb_ref, bb_ref, g_ref, be_ref,
                 out_ref):
    h = p_ref[0] + p_ref[1]
    h = jnp.maximum(
        jnp.dot(h, Wa_ref[...], preferred_element_type=jnp.float32) + ba_ref[...], 0.0)
    h = jnp.dot(h, Wb_ref[...], preferred_element_type=jnp.float32) + bb_ref[...]
    mu = jnp.mean(h, axis=0, keepdims=True)
    var = jnp.mean((h - mu) * (h - mu), axis=0, keepdims=True)
    h = (h - mu) * lax.rsqrt(var + 1e-5) * g_ref[...] + be_ref[...]
    out_ref[...] = jnp.maximum(h, 0.0)


def _mlp_bn_res_body(p_ref, Wa_ref, ba_ref, Wb_ref, bb_ref, g_ref, be_ref,
                     x0_ref, out_ref):
    h = p_ref[0] + p_ref[1]
    h = jnp.maximum(
        jnp.dot(h, Wa_ref[...], preferred_element_type=jnp.float32) + ba_ref[...], 0.0)
    h = jnp.dot(h, Wb_ref[...], preferred_element_type=jnp.float32) + bb_ref[...]
    mu = jnp.mean(h, axis=0, keepdims=True)
    var = jnp.mean((h - mu) * (h - mu), axis=0, keepdims=True)
    h = (h - mu) * lax.rsqrt(var + 1e-5) * g_ref[...] + be_ref[...]
    out_ref[...] = (jnp.maximum(h, 0.0) + x0_ref[...]) * np.float32(1.0 / np.sqrt(2.0))


def _mlp_bn(p, Wa, ba, Wb, bb, g, be):
    return pl.pallas_call(
        _mlp_bn_body,
        out_shape=jax.ShapeDtypeStruct((N, D), jnp.float32),
    )(p, Wa, ba.reshape(1, D), Wb, bb.reshape(1, D), g.reshape(1, D),
      be.reshape(1, D))


def _mlp_bn_res(p, Wa, ba, Wb, bb, g, be, x0):
    return pl.pallas_call(
        _mlp_bn_res_body,
        out_shape=jax.ShapeDtypeStruct((N, D), jnp.float32),
    )(p, Wa, ba.reshape(1, D), Wb, bb.reshape(1, D), g.reshape(1, D),
      be.reshape(1, D), x0)


def kernel(x, edge_index, W1, b1, W2, b2, W3, b3, W4, b4, g1, be1, g2, be2):
    src = edge_index[0].astype(jnp.int32)
    dst = edge_index[1].astype(jnp.int32)
    pad_i = jnp.arange(E_PAD - E, dtype=jnp.int32)
    src3d = jnp.concatenate([src, pad_i % N]).reshape(NW, CPW, CHUNK)
    dst3d = jnp.concatenate([dst, N + pad_i % NTRASH]).reshape(NW, CPW, CHUNK)
    zeros = jnp.zeros((ROWS_PER_SUB, D), jnp.float32)

    p1 = _seg_sum(src3d, dst3d, x, zeros)
    h1 = _mlp_bn(p1, W1, b1, W2, b2, g1, be1)
    p2 = _seg_sum(src3d, dst3d, h1, zeros)
    return _mlp_bn_res(p2, W3, b3, W4, b4, g2, be2, x)


# speedup vs baseline: 1.0726x; 1.0037x over previous
import functools

import jax
import jax.numpy as jnp
import numpy as np
from jax import lax
from jax.experimental import pallas as pl
from jax.experimental.pallas import tpu as pltpu
from jax.experimental.pallas import tpu_sc as plsc

N = 10000
E = 320000
D = 128

NC = 2
NS = 16
NW = NC * NS

CHUNK = 120
CPW = 84
E_PAD = NW * CPW * CHUNK
NBUF = 3
NIDX = 6
GL = 2
KI = NIDX - NBUF + GL
NTRASH = 96
ACC_ROWS = N + NTRASH

ROWS_PER_SUB = 624
TAIL_ROWS = N - NS * ROWS_PER_SUB


def _seg_sum_body(src_hbm, dst_hbm, x_hbm, zeros_hbm, out_hbm,
                  idx_s, idx_d, rows, acc, semi, semg, sems):
    c = lax.axis_index("c")
    s = lax.axis_index("s")
    w = s * NC + c

    @pl.when(c == 0)
    def _():
        pltpu.sync_copy(x_hbm.at[pl.ds(s * ROWS_PER_SUB, ROWS_PER_SUB)],
                        acc.at[pl.ds(s * ROWS_PER_SUB, ROWS_PER_SUB)])

    @pl.when(c == 1)
    def _():
        pltpu.sync_copy(zeros_hbm,
                        acc.at[pl.ds(s * ROWS_PER_SUB, ROWS_PER_SUB)])

    @pl.when((s == 0) & (c == 0))
    def _():
        pltpu.sync_copy(x_hbm.at[pl.ds(NS * ROWS_PER_SUB, TAIL_ROWS)],
                        acc.at[pl.ds(NS * ROWS_PER_SUB, TAIL_ROWS)])

    @pl.when((s == 0) & (c == 1))
    def _():
        pltpu.sync_copy(zeros_hbm.at[pl.ds(0, TAIL_ROWS)],
                        acc.at[pl.ds(NS * ROWS_PER_SUB, TAIL_ROWS)])

    @pl.when(s == 1)
    def _():
        pltpu.sync_copy(zeros_hbm.at[pl.ds(0, NTRASH)],
                        acc.at[pl.ds(N, NTRASH)])

    plsc.subcore_barrier()

    def start_idx(j, q):
        pltpu.async_copy(src_hbm.at[w, j], idx_s.at[q], semi[q])
        pltpu.async_copy(dst_hbm.at[w, j], idx_d.at[q], semi[q])

    def wait_idx(j, q):
        pltpu.make_async_copy(src_hbm.at[w, j], idx_s.at[q], semi[q]).wait()
        pltpu.make_async_copy(dst_hbm.at[w, j], idx_d.at[q], semi[q]).wait()

    def start_gather(q, b):
        pltpu.async_copy(x_hbm.at[idx_s.at[q]], rows.at[b], semg[b])

    def wait_gather(q, b):
        pltpu.make_async_copy(x_hbm.at[idx_s.at[q]], rows.at[b],
                              semg[b]).wait()

    def start_scatter(q, b):
        pltpu.async_copy(rows.at[b], acc.at[idx_d.at[q]], sems[b], add=True)

    def wait_scatter(q, b):
        pltpu.make_async_copy(rows.at[b], acc.at[idx_d.at[q]],
                              sems[b]).wait()

    for q in range(KI):
        start_idx(q, q)
    for q in range(GL):
        wait_idx(q, q)
        start_gather(q, q)

    def slot(j, u, dyn):
        if dyn or j + GL < CPW:
            wait_idx(j + GL, (u + GL) % NIDX)
            if dyn or j - (NBUF - GL) >= 0:
                wait_scatter((u - (NBUF - GL)) % NIDX, (u + GL) % NBUF)
            start_gather((u + GL) % NIDX, (u + GL) % NBUF)
        if dyn or j + KI < CPW:
            start_idx(j + KI, (u + KI) % NIDX)
        wait_gather(u % NIDX, u % NBUF)
        start_scatter(u % NIDX, u % NBUF)

    for u in range(NIDX):
        slot(u, u, dyn=False)

    def loop_body(p, carry):
        j = p * NIDX
        for u in range(NIDX):
            slot(j + u, u, dyn=True)
        return carry

    lax.fori_loop(1, CPW // NIDX - 1, loop_body, 0)

    for u in range(NIDX):
        slot(CPW - NIDX + u, u, dyn=False)

    for j in range(CPW - NBUF, CPW):
        wait_scatter(j % NIDX, j % NBUF)

    plsc.subcore_barrier()
    pltpu.sync_copy(acc.at[pl.ds(s * ROWS_PER_SUB, ROWS_PER_SUB)],
                    out_hbm.at[c, pl.ds(s * ROWS_PER_SUB, ROWS_PER_SUB)])

    @pl.when(s == 0)
    def _():
        pltpu.sync_copy(acc.at[pl.ds(NS * ROWS_PER_SUB, TAIL_ROWS)],
                        out_hbm.at[c, pl.ds(NS * ROWS_PER_SUB, TAIL_ROWS)])


def _seg_sum(src3d, dst3d, x, zeros):
    mesh = plsc.VectorSubcoreMesh(core_axis_name="c", subcore_axis_name="s",
                                  num_cores=NC, num_subcores=NS)
    f = pl.kernel(
        _seg_sum_body,
        out_type=jax.ShapeDtypeStruct((NC, N, D), jnp.float32),
        mesh=mesh,
        scratch_types=[
            pltpu.VMEM((NIDX, CHUNK), jnp.int32),
            pltpu.VMEM((NIDX, CHUNK), jnp.int32),
            pltpu.VMEM((NBUF, CHUNK, D), jnp.float32),
            pltpu.VMEM_SHARED((ACC_ROWS, D), jnp.float32),
            [pltpu.SemaphoreType.DMA] * NIDX,
            [pltpu.SemaphoreType.DMA] * NBUF,
            [pltpu.SemaphoreType.DMA] * NBUF,
        ],
    )
    return f(src3d, dst3d, x, zeros)


def _mlp_bn_body(p_ref, Wa_ref, ba_ref, Wb_ref, bb_ref, g_ref, be_ref,
                 out_ref):
    h = p_ref[0] + p_ref[1]
    h = jnp.maximum(
        jnp.dot(h, Wa_ref[...], preferred_element_type=jnp.float32) + ba_ref[...], 0.0)
    h = jnp.dot(h, Wb_ref[...], preferred_element_type=jnp.float32) + bb_ref[...]
    mu = jnp.mean(h, axis=0, keepdims=True)
    var = jnp.mean((h - mu) * (h - mu), axis=0, keepdims=True)
    h = (h - mu) * lax.rsqrt(var + 1e-5) * g_ref[...] + be_ref[...]
    out_ref[...] = jnp.maximum(h, 0.0)


def _mlp_bn_res_body(p_ref, Wa_ref, ba_ref, Wb_ref, bb_ref, g_ref, be_ref,
                     x0_ref, out_ref):
    h = p_ref[0] + p_ref[1]
    h = jnp.maximum(
        jnp.dot(h, Wa_ref[...], preferred_element_type=jnp.float32) + ba_ref[...], 0.0)
    h = jnp.dot(h, Wb_ref[...], preferred_element_type=jnp.float32) + bb_ref[...]
    mu = jnp.mean(h, axis=0, keepdims=True)
    var = jnp.mean((h - mu) * (h - mu), axis=0, keepdims=True)
    h = (h - mu) * lax.rsqrt(var + 1e-5) * g_ref[...] + be_ref[...]
    out_ref[...] = (jnp.maximum(h, 0.0) + x0_ref[...]) * np.float32(1.0 / np.sqrt(2.0))


def _mlp_bn(p, Wa, ba, Wb, bb, g, be):
    return pl.pallas_call(
        _mlp_bn_body,
        out_shape=jax.ShapeDtypeStruct((N, D), jnp.float32),
    )(p, Wa, ba.reshape(1, D), Wb, bb.reshape(1, D), g.reshape(1, D),
      be.reshape(1, D))


def _mlp_bn_res(p, Wa, ba, Wb, bb, g, be, x0):
    return pl.pallas_call(
        _mlp_bn_res_body,
        out_shape=jax.ShapeDtypeStruct((N, D), jnp.float32),
    )(p, Wa, ba.reshape(1, D), Wb, bb.reshape(1, D), g.reshape(1, D),
      be.reshape(1, D), x0)


def kernel(x, edge_index, W1, b1, W2, b2, W3, b3, W4, b4, g1, be1, g2, be2):
    src = edge_index[0].astype(jnp.int32)
    dst = edge_index[1].astype(jnp.int32)
    pad_i = jnp.arange(E_PAD - E, dtype=jnp.int32)
    src3d = jnp.concatenate([src, pad_i % N]).reshape(NW, CPW, CHUNK)
    dst3d = jnp.concatenate([dst, N + pad_i % NTRASH]).reshape(NW, CPW, CHUNK)
    zeros = jnp.zeros((ROWS_PER_SUB, D), jnp.float32)

    p1 = _seg_sum(src3d, dst3d, x, zeros)
    h1 = _mlp_bn(p1, W1, b1, W2, b2, g1, be1)
    p2 = _seg_sum(src3d, dst3d, h1, zeros)
    return _mlp_bn_res(p2, W3, b3, W4, b4, g2, be2, x)
